# Initial kernel scaffold; baseline (speedup 1.0000x reference)
import functools

import jax
import jax.numpy as jnp
from jax import lax
from jax.experimental import pallas as pl
from jax.experimental.pallas import tpu as pltpu
from jax.experimental.pallas import tpu_sc as plsc

_NUM_OPTIONS = 16
_NUM_ACTIONS = 32
_D_SMALL = _NUM_OPTIONS
_D_IOP = _NUM_OPTIONS * _NUM_ACTIONS
_NC, _NS = 2, 16
_NW = _NC * _NS


@functools.lru_cache(maxsize=None)
def _build(B):
    b_per_w = B // _NW
    mesh = plsc.VectorSubcoreMesh(core_axis_name="c", subcore_axis_name="s")

    def body(obs_hbm, beta_hbm, iop_hbm, poo_hbm, q_hbm,
             beta_o, iop_o, poo_o, q_o,
             idx_v, beta_v, iop_v, poo_v, q_v, sem):
        wid = lax.axis_index("s") * _NC + lax.axis_index("c")
        base = wid * b_per_w
        pltpu.sync_copy(obs_hbm.at[pl.ds(base, b_per_w)], idx_v)
        c1 = pltpu.async_copy(beta_hbm.at[idx_v], beta_v, sem)
        c2 = pltpu.async_copy(iop_hbm.at[idx_v], iop_v, sem)
        c3 = pltpu.async_copy(poo_hbm.at[idx_v], poo_v, sem)
        c4 = pltpu.async_copy(q_hbm.at[idx_v], q_v, sem)
        c1.wait()

        @pl.loop(0, b_per_w)
        def _(i):
            row = beta_v[i, :]
            beta_v[i, :] = 1.0 / (1.0 + jnp.exp(-row))

        pltpu.sync_copy(beta_v, beta_o.at[pl.ds(base, b_per_w)])
        c2.wait()
        pltpu.sync_copy(iop_v, iop_o.at[pl.ds(base, b_per_w)])
        c3.wait()
        pltpu.sync_copy(poo_v, poo_o.at[pl.ds(base, b_per_w)])
        c4.wait()
        pltpu.sync_copy(q_v, q_o.at[pl.ds(base, b_per_w)])

    return pl.kernel(
        body,
        out_type=[
            jax.ShapeDtypeStruct((B, _D_SMALL), jnp.float32),
            jax.ShapeDtypeStruct((B, _D_IOP), jnp.float32),
            jax.ShapeDtypeStruct((B, _D_SMALL), jnp.float32),
            jax.ShapeDtypeStruct((B, _D_SMALL), jnp.float32),
        ],
        mesh=mesh,
        scratch_types=[
            pltpu.VMEM((b_per_w,), jnp.int32),
            pltpu.VMEM((b_per_w, _D_SMALL), jnp.float32),
            pltpu.VMEM((b_per_w, _D_IOP), jnp.float32),
            pltpu.VMEM((b_per_w, _D_SMALL), jnp.float32),
            pltpu.VMEM((b_per_w, _D_SMALL), jnp.float32),
            pltpu.SemaphoreType.DMA,
        ],
    )


@jax.jit
def kernel(obs, beta, iop, poo, q):
    B = obs.shape[0]
    V = iop.shape[0]
    iop2 = iop.reshape(V, _D_IOP)
    beta_o, iop_o, poo_o, q_o = _build(B)(obs, beta, iop2, poo, q)
    return (
        beta_o.reshape(-1),
        iop_o.reshape(-1, _NUM_ACTIONS),
        poo_o.reshape(-1),
        q_o.reshape(-1),
    )


# speedup vs baseline: 1.5833x; 1.5833x over previous
import functools

import jax
import jax.numpy as jnp
from jax import lax
from jax.experimental import pallas as pl
from jax.experimental.pallas import tpu as pltpu
from jax.experimental.pallas import tpu_sc as plsc

_NUM_OPTIONS = 16
_NUM_ACTIONS = 32
_D_SMALL = _NUM_OPTIONS
_D_IOP = _NUM_OPTIONS * _NUM_ACTIONS
_NC, _NS = 2, 16
_NW = _NC * _NS


@functools.lru_cache(maxsize=None)
def _build(B):
    b_per_w = B // _NW
    mesh = plsc.VectorSubcoreMesh(core_axis_name="c", subcore_axis_name="s")

    def body(obs_hbm, beta_hbm, iop_hbm, poo_hbm, q_hbm,
             beta_o, iop_o, poo_o, q_o,
             idx_v, beta_v, iop_v, poo_v, q_v, sem):
        wid = lax.axis_index("s") * _NC + lax.axis_index("c")
        base = wid * b_per_w
        pltpu.sync_copy(obs_hbm.at[pl.ds(base, b_per_w)], idx_v)
        c1 = pltpu.async_copy(beta_hbm.at[idx_v], beta_v, sem)
        c2 = pltpu.async_copy(iop_hbm.at[idx_v], iop_v, sem)
        c3 = pltpu.async_copy(poo_hbm.at[idx_v], poo_v, sem)
        c4 = pltpu.async_copy(q_hbm.at[idx_v], q_v, sem)
        c1.wait()

        @pl.loop(0, b_per_w)
        def _(i):
            row = beta_v[i, :]
            beta_v[i, :] = 1.0 / (1.0 + jnp.exp(-row))

        pltpu.sync_copy(beta_v, beta_o.at[pl.ds(base, b_per_w)])
        c2.wait()
        pltpu.sync_copy(iop_v, iop_o.at[pl.ds(base, b_per_w)])
        c3.wait()
        pltpu.sync_copy(poo_v, poo_o.at[pl.ds(base, b_per_w)])
        c4.wait()
        pltpu.sync_copy(q_v, q_o.at[pl.ds(base, b_per_w)])

    return pl.kernel(
        body,
        compiler_params=pltpu.CompilerParams(use_tc_tiling_on_sc=False),
        out_type=[
            jax.ShapeDtypeStruct((B, _D_SMALL), jnp.float32),
            jax.ShapeDtypeStruct((B, _D_IOP), jnp.float32),
            jax.ShapeDtypeStruct((B, _D_SMALL), jnp.float32),
            jax.ShapeDtypeStruct((B, _D_SMALL), jnp.float32),
        ],
        mesh=mesh,
        scratch_types=[
            pltpu.VMEM((b_per_w,), jnp.int32),
            pltpu.VMEM((b_per_w, _D_SMALL), jnp.float32),
            pltpu.VMEM((b_per_w, _D_IOP), jnp.float32),
            pltpu.VMEM((b_per_w, _D_SMALL), jnp.float32),
            pltpu.VMEM((b_per_w, _D_SMALL), jnp.float32),
            pltpu.SemaphoreType.DMA,
        ],
    )


@jax.jit
def kernel(obs, beta, iop, poo, q):
    B = obs.shape[0]
    V = iop.shape[0]
    iop2 = iop.reshape(V, _D_IOP)
    beta_o, iop_o, poo_o, q_o = _build(B)(obs, beta, iop2, poo, q)
    return (
        beta_o.reshape(-1),
        iop_o.reshape(-1, _NUM_ACTIONS),
        poo_o.reshape(-1),
        q_o.reshape(-1),
    )


# speedup vs baseline: 2.0259x; 1.2796x over previous
import functools

import jax
import jax.numpy as jnp
from jax import lax
from jax.experimental import pallas as pl
from jax.experimental.pallas import tpu as pltpu
from jax.experimental.pallas import tpu_sc as plsc

_NUM_OPTIONS = 16
_NUM_ACTIONS = 32
_D_SMALL = _NUM_OPTIONS
_D_IOP = _NUM_OPTIONS * _NUM_ACTIONS
_NC, _NS = 2, 16
_NW = _NC * _NS

_mesh = plsc.VectorSubcoreMesh(core_axis_name="c", subcore_axis_name="s")


@functools.lru_cache(maxsize=None)
def _build_iop(B, V):
    b_per_w = B // _NW

    def body(obs_hbm, iop_hbm, iop_o, idx_v, iop_v, sem):
        wid = lax.axis_index("s") * _NC + lax.axis_index("c")
        base = wid * b_per_w
        pltpu.sync_copy(obs_hbm.at[pl.ds(base, b_per_w)], idx_v)
        pltpu.async_copy(iop_hbm.at[idx_v], iop_v, sem).wait()
        pltpu.sync_copy(iop_v, iop_o.at[pl.ds(base, b_per_w)])

    return pl.kernel(
        body,
        out_type=[jax.ShapeDtypeStruct((B, _D_IOP), jnp.float32)],
        mesh=_mesh,
        scratch_types=[
            pltpu.VMEM((b_per_w,), jnp.int32),
            pltpu.VMEM((b_per_w, _D_IOP), jnp.float32),
            pltpu.SemaphoreType.DMA,
        ],
    )


@functools.lru_cache(maxsize=None)
def _build_small(B, V):
    b_per_w = B // _NW

    def body(obs_hbm, beta_hbm, poo_hbm, q_hbm,
             beta_o, poo_o, q_o,
             idx_v, beta_v, poo_v, q_v, sem):
        wid = lax.axis_index("s") * _NC + lax.axis_index("c")
        base = wid * b_per_w
        pltpu.sync_copy(obs_hbm.at[pl.ds(base, b_per_w)], idx_v)
        c1 = pltpu.async_copy(beta_hbm.at[idx_v], beta_v, sem)
        c2 = pltpu.async_copy(poo_hbm.at[idx_v], poo_v, sem)
        c3 = pltpu.async_copy(q_hbm.at[idx_v], q_v, sem)
        c1.wait()

        @pl.loop(0, b_per_w)
        def _(i):
            row = beta_v[i, :]
            beta_v[i, :] = 1.0 / (1.0 + jnp.exp(-row))

        pltpu.sync_copy(beta_v, beta_o.at[pl.ds(base, b_per_w)])
        c2.wait()
        pltpu.sync_copy(poo_v, poo_o.at[pl.ds(base, b_per_w)])
        c3.wait()
        pltpu.sync_copy(q_v, q_o.at[pl.ds(base, b_per_w)])

    return pl.kernel(
        body,
        compiler_params=pltpu.CompilerParams(use_tc_tiling_on_sc=False),
        out_type=[
            jax.ShapeDtypeStruct((B, _D_SMALL), jnp.float32),
            jax.ShapeDtypeStruct((B, _D_SMALL), jnp.float32),
            jax.ShapeDtypeStruct((B, _D_SMALL), jnp.float32),
        ],
        mesh=_mesh,
        scratch_types=[
            pltpu.VMEM((b_per_w,), jnp.int32),
            pltpu.VMEM((b_per_w, _D_SMALL), jnp.float32),
            pltpu.VMEM((b_per_w, _D_SMALL), jnp.float32),
            pltpu.VMEM((b_per_w, _D_SMALL), jnp.float32),
            pltpu.SemaphoreType.DMA,
        ],
    )


@jax.jit
def kernel(obs, beta, iop, poo, q):
    B = obs.shape[0]
    V = iop.shape[0]
    iop2 = iop.reshape(V, _D_IOP)
    (iop_o,) = _build_iop(B, V)(obs, iop2)
    beta_o, poo_o, q_o = _build_small(B, V)(obs, beta, poo, q)
    return (
        beta_o.reshape(-1),
        iop_o.reshape(-1, _NUM_ACTIONS),
        poo_o.reshape(-1),
        q_o.reshape(-1),
    )
